# baseline (speedup 1.0000x reference)
import functools

import jax
import jax.numpy as jnp
from jax import lax
from jax.experimental import pallas as pl
from jax.experimental.pallas import tpu as pltpu
from jax.experimental.pallas import tpu_sc as plsc

BATCH = 16384
K_EMB = 32
NC = 2
NS = 16
NW = NC * NS
BPW = BATCH // NW

BLK = 2048


RING = 10
LANES = 128


def _sc_gather_t(ut_t, it_t, u, i):
    mesh = plsc.VectorSubcoreMesh(
        core_axis_name="c", subcore_axis_name="s", num_cores=NC)

    @functools.partial(
        pl.kernel,
        mesh=mesh,
        compiler_params=pltpu.CompilerParams(needs_layout_passes=False),
        out_type=(
            jax.ShapeDtypeStruct((K_EMB, BATCH), jnp.float32),
            jax.ShapeDtypeStruct((K_EMB, BATCH), jnp.float32),
        ),
        scratch_types=[
            pltpu.VMEM((BPW,), jnp.int32),
            pltpu.VMEM((BPW,), jnp.int32),
            pltpu.VMEM((RING, K_EMB, LANES), jnp.float32),
            pltpu.VMEM((RING, K_EMB, LANES), jnp.float32),
            pltpu.VMEM((K_EMB, BPW), jnp.float32),
            pltpu.VMEM((K_EMB, BPW), jnp.float32),
            pltpu.SemaphoreType.DMA((RING,)),
            pltpu.SemaphoreType.DMA((RING,)),
        ],
    )
    def gather_k(ut_hbm, it_hbm, u_hbm, i_hbm, ue_out, ie_out,
                 uidx_v, iidx_v, uring, iring, ucols, icols, usem, isem):
        wid = lax.axis_index("s") * NC + lax.axis_index("c")
        base = wid * BPW
        pltpu.sync_copy(u_hbm.at[pl.ds(base, BPW)], uidx_v)
        pltpu.sync_copy(i_hbm.at[pl.ds(base, BPW)], iidx_v)
        row0 = lax.iota(jnp.int32, 16)
        row1 = row0 + 16

        def fire(cu, ci, slot):
            tu = pl.multiple_of((cu // LANES) * LANES, LANES)
            ti = pl.multiple_of((ci // LANES) * LANES, LANES)
            pltpu.make_async_copy(
                ut_hbm.at[:, pl.ds(tu, LANES)], uring.at[slot],
                usem.at[slot]).start()
            pltpu.make_async_copy(
                it_hbm.at[:, pl.ds(ti, LANES)], iring.at[slot],
                isem.at[slot]).start()

        def extract(cu, ci, bpos, slot):
            pltpu.make_async_copy(
                ut_hbm.at[:, pl.ds(0, LANES)], uring.at[slot],
                usem.at[slot]).wait()
            pltpu.make_async_copy(
                it_hbm.at[:, pl.ds(0, LANES)], iring.at[slot],
                isem.at[slot]).wait()
            lu = jnp.full((16,), lax.rem(cu, LANES), jnp.int32)
            li = jnp.full((16,), lax.rem(ci, LANES), jnp.int32)
            bcol = jnp.full((16,), bpos, jnp.int32)
            for rr in (row0, row1):
                vu = plsc.load_gather(uring.at[slot], [rr, lu])
                plsc.store_scatter(ucols, [rr, bcol], vu)
                vi = plsc.load_gather(iring.at[slot], [rr, li])
                plsc.store_scatter(icols, [rr, bcol], vi)

        def body(t, carry):
            t0 = t * 16
            uv = uidx_v[pl.ds(t0, 16)]
            iv = iidx_v[pl.ds(t0, 16)]
            tp = jnp.maximum(t - 1, 0) * 16
            up = uidx_v[pl.ds(tp, 16)]
            ip = iidx_v[pl.ds(tp, 16)]
            for j in range(16):
                eslot = lax.rem(t0 + j - RING, RING)
                fslot = lax.rem(t0 + j, RING)
                lp = (j + 16 - RING) % 16
                if j < RING:
                    @pl.when(t > 0)
                    def _(up=up, ip=ip, lp=lp, eslot=eslot, t0=t0, j=j):
                        extract(up[lp], ip[lp], t0 + j - RING, eslot)
                else:
                    extract(uv[lp], iv[lp], t0 + j - RING, eslot)
                fire(uv[j], iv[j], fslot)
            return carry

        lax.fori_loop(0, BPW // 16, body, 0)
        tl = (BPW // 16 - 1) * 16
        uv = uidx_v[pl.ds(tl, 16)]
        iv = iidx_v[pl.ds(tl, 16)]
        for j in range(16 - RING, 16):
            extract(uv[j], iv[j], tl + j, lax.rem(tl + j, RING))
        pltpu.sync_copy(ucols, ue_out.at[:, pl.ds(base, BPW)])
        pltpu.sync_copy(icols, ie_out.at[:, pl.ds(base, BPW)])

    return gather_k(ut_t, it_t, u, i)


def _mlp_body(ue_ref, ie_ref, g_ref, s_ref, w1u_ref, w1i_ref, w1g_ref,
              w1s_ref, b1_ref, w2_ref, b2_ref, w3_ref, b3_ref, out_ref):
    bf = jnp.bfloat16
    f32 = jnp.float32

    def dot(a, b):
        return jnp.dot(a.astype(bf), b.astype(bf), preferred_element_type=f32)

    x1 = dot(w1u_ref[:], ue_ref[:])
    x1 = x1 + dot(w1i_ref[:], ie_ref[:])
    x1 = x1 + dot(w1g_ref[:], g_ref[:])
    x1 = x1 + dot(w1s_ref[:], s_ref[:])
    h1 = jnp.maximum(x1 + b1_ref[:], 0.0)
    h2 = jnp.maximum(dot(w2_ref[:], h1) + b2_ref[:], 0.0)
    out_ref[:] = dot(w3_ref[:], h2) + b3_ref[:]


def kernel(u, i, g, s, user_table, item_table, W1, b1, W2, b2, W3, b3):
    ue_t, ie_t = _sc_gather_t(user_table.T, item_table.T,
                              u.astype(jnp.int32), i.astype(jnp.int32))

    g_t = g.T
    s_t = s[None, :]
    w1u_t = W1[:K_EMB].T
    w1i_t = W1[K_EMB:2 * K_EMB].T
    w1g_t = W1[2 * K_EMB:2 * K_EMB + 19].T
    w1s_t = W1[2 * K_EMB + 19:].T
    w2_t = W2.T
    w3_t = W3.T
    b1c = b1[:, None]
    b2c = b2[:, None]
    b3c = b3[:, None]

    grid = (BATCH // BLK,)
    out2d = pl.pallas_call(
        _mlp_body,
        grid=grid,
        in_specs=[
            pl.BlockSpec((K_EMB, BLK), lambda b: (0, b)),
            pl.BlockSpec((K_EMB, BLK), lambda b: (0, b)),
            pl.BlockSpec((19, BLK), lambda b: (0, b)),
            pl.BlockSpec((1, BLK), lambda b: (0, b)),
            pl.BlockSpec(w1u_t.shape, lambda b: (0, 0)),
            pl.BlockSpec(w1i_t.shape, lambda b: (0, 0)),
            pl.BlockSpec(w1g_t.shape, lambda b: (0, 0)),
            pl.BlockSpec(w1s_t.shape, lambda b: (0, 0)),
            pl.BlockSpec(b1c.shape, lambda b: (0, 0)),
            pl.BlockSpec(w2_t.shape, lambda b: (0, 0)),
            pl.BlockSpec(b2c.shape, lambda b: (0, 0)),
            pl.BlockSpec(w3_t.shape, lambda b: (0, 0)),
            pl.BlockSpec(b3c.shape, lambda b: (0, 0)),
        ],
        out_specs=pl.BlockSpec((1, BLK), lambda b: (0, b)),
        out_shape=jax.ShapeDtypeStruct((1, BATCH), jnp.float32),
    )(ue_t, ie_t, g_t, s_t, w1u_t, w1i_t, w1g_t, w1s_t, b1c, w2_t, b2c,
      w3_t, b3c)
    return out2d[0]


# speedup vs baseline: 1.0095x; 1.0095x over previous
import functools

import jax
import jax.numpy as jnp
from jax import lax
from jax.experimental import pallas as pl
from jax.experimental.pallas import tpu as pltpu
from jax.experimental.pallas import tpu_sc as plsc

BATCH = 16384
K_EMB = 32
NC = 2
NS = 16
NW = NC * NS
BPW = BATCH // NW

BLK = 4096


RING = 10
LANES = 128


def _sc_gather_t(ut_t, it_t, u, i):
    mesh = plsc.VectorSubcoreMesh(
        core_axis_name="c", subcore_axis_name="s", num_cores=NC)

    @functools.partial(
        pl.kernel,
        mesh=mesh,
        compiler_params=pltpu.CompilerParams(needs_layout_passes=False),
        out_type=(
            jax.ShapeDtypeStruct((K_EMB, BATCH), jnp.float32),
            jax.ShapeDtypeStruct((K_EMB, BATCH), jnp.float32),
        ),
        scratch_types=[
            pltpu.VMEM((BPW,), jnp.int32),
            pltpu.VMEM((BPW,), jnp.int32),
            pltpu.VMEM((RING, K_EMB, LANES), jnp.float32),
            pltpu.VMEM((RING, K_EMB, LANES), jnp.float32),
            pltpu.VMEM((K_EMB, BPW), jnp.float32),
            pltpu.VMEM((K_EMB, BPW), jnp.float32),
            pltpu.SemaphoreType.DMA((RING,)),
            pltpu.SemaphoreType.DMA((RING,)),
        ],
    )
    def gather_k(ut_hbm, it_hbm, u_hbm, i_hbm, ue_out, ie_out,
                 uidx_v, iidx_v, uring, iring, ucols, icols, usem, isem):
        wid = lax.axis_index("s") * NC + lax.axis_index("c")
        base = wid * BPW
        pltpu.sync_copy(u_hbm.at[pl.ds(base, BPW)], uidx_v)
        pltpu.sync_copy(i_hbm.at[pl.ds(base, BPW)], iidx_v)
        row0 = lax.iota(jnp.int32, 16)
        row1 = row0 + 16

        def fire(cu, ci, slot):
            tu = pl.multiple_of((cu // LANES) * LANES, LANES)
            ti = pl.multiple_of((ci // LANES) * LANES, LANES)
            pltpu.make_async_copy(
                ut_hbm.at[:, pl.ds(tu, LANES)], uring.at[slot],
                usem.at[slot]).start()
            pltpu.make_async_copy(
                it_hbm.at[:, pl.ds(ti, LANES)], iring.at[slot],
                isem.at[slot]).start()

        def extract(cu, ci, bpos, slot):
            pltpu.make_async_copy(
                ut_hbm.at[:, pl.ds(0, LANES)], uring.at[slot],
                usem.at[slot]).wait()
            pltpu.make_async_copy(
                it_hbm.at[:, pl.ds(0, LANES)], iring.at[slot],
                isem.at[slot]).wait()
            lu = jnp.full((16,), lax.rem(cu, LANES), jnp.int32)
            li = jnp.full((16,), lax.rem(ci, LANES), jnp.int32)
            bcol = jnp.full((16,), bpos, jnp.int32)
            for rr in (row0, row1):
                vu = plsc.load_gather(uring.at[slot], [rr, lu])
                plsc.store_scatter(ucols, [rr, bcol], vu)
                vi = plsc.load_gather(iring.at[slot], [rr, li])
                plsc.store_scatter(icols, [rr, bcol], vi)

        def body(t, carry):
            t0 = t * 16
            uv = uidx_v[pl.ds(t0, 16)]
            iv = iidx_v[pl.ds(t0, 16)]
            tp = jnp.maximum(t - 1, 0) * 16
            up = uidx_v[pl.ds(tp, 16)]
            ip = iidx_v[pl.ds(tp, 16)]
            for j in range(16):
                eslot = lax.rem(t0 + j - RING, RING)
                fslot = lax.rem(t0 + j, RING)
                lp = (j + 16 - RING) % 16
                if j < RING:
                    @pl.when(t > 0)
                    def _(up=up, ip=ip, lp=lp, eslot=eslot, t0=t0, j=j):
                        extract(up[lp], ip[lp], t0 + j - RING, eslot)
                else:
                    extract(uv[lp], iv[lp], t0 + j - RING, eslot)
                fire(uv[j], iv[j], fslot)
            return carry

        lax.fori_loop(0, BPW // 16, body, 0)
        tl = (BPW // 16 - 1) * 16
        uv = uidx_v[pl.ds(tl, 16)]
        iv = iidx_v[pl.ds(tl, 16)]
        for j in range(16 - RING, 16):
            extract(uv[j], iv[j], tl + j, lax.rem(tl + j, RING))
        pltpu.sync_copy(ucols, ue_out.at[:, pl.ds(base, BPW)])
        pltpu.sync_copy(icols, ie_out.at[:, pl.ds(base, BPW)])

    return gather_k(ut_t, it_t, u, i)


def _mlp_body(ue_ref, ie_ref, g_ref, s_ref, w1u_ref, w1i_ref, w1g_ref,
              w1s_ref, b1_ref, w2_ref, b2_ref, w3_ref, b3_ref, out_ref):
    bf = jnp.bfloat16
    f32 = jnp.float32

    def dot(a, b):
        return jnp.dot(a.astype(bf), b.astype(bf), preferred_element_type=f32)

    x1 = dot(w1u_ref[:], ue_ref[:])
    x1 = x1 + dot(w1i_ref[:], ie_ref[:])
    x1 = x1 + dot(w1g_ref[:], g_ref[:])
    x1 = x1 + dot(w1s_ref[:], s_ref[:])
    h1 = jnp.maximum(x1 + b1_ref[:], 0.0)
    h2 = jnp.maximum(dot(w2_ref[:], h1) + b2_ref[:], 0.0)
    out_ref[:] = dot(w3_ref[:], h2) + b3_ref[:]


def kernel(u, i, g, s, user_table, item_table, W1, b1, W2, b2, W3, b3):
    ue_t, ie_t = _sc_gather_t(user_table.T, item_table.T,
                              u.astype(jnp.int32), i.astype(jnp.int32))

    g_t = g.T
    s_t = s[None, :]
    w1u_t = W1[:K_EMB].T
    w1i_t = W1[K_EMB:2 * K_EMB].T
    w1g_t = W1[2 * K_EMB:2 * K_EMB + 19].T
    w1s_t = W1[2 * K_EMB + 19:].T
    w2_t = W2.T
    w3_t = W3.T
    b1c = b1[:, None]
    b2c = b2[:, None]
    b3c = b3[:, None]

    grid = (BATCH // BLK,)
    out2d = pl.pallas_call(
        _mlp_body,
        grid=grid,
        in_specs=[
            pl.BlockSpec((K_EMB, BLK), lambda b: (0, b)),
            pl.BlockSpec((K_EMB, BLK), lambda b: (0, b)),
            pl.BlockSpec((19, BLK), lambda b: (0, b)),
            pl.BlockSpec((1, BLK), lambda b: (0, b)),
            pl.BlockSpec(w1u_t.shape, lambda b: (0, 0)),
            pl.BlockSpec(w1i_t.shape, lambda b: (0, 0)),
            pl.BlockSpec(w1g_t.shape, lambda b: (0, 0)),
            pl.BlockSpec(w1s_t.shape, lambda b: (0, 0)),
            pl.BlockSpec(b1c.shape, lambda b: (0, 0)),
            pl.BlockSpec(w2_t.shape, lambda b: (0, 0)),
            pl.BlockSpec(b2c.shape, lambda b: (0, 0)),
            pl.BlockSpec(w3_t.shape, lambda b: (0, 0)),
            pl.BlockSpec(b3c.shape, lambda b: (0, 0)),
        ],
        out_specs=pl.BlockSpec((1, BLK), lambda b: (0, b)),
        out_shape=jax.ShapeDtypeStruct((1, BATCH), jnp.float32),
    )(ue_t, ie_t, g_t, s_t, w1u_t, w1i_t, w1g_t, w1s_t, b1c, w2_t, b2c,
      w3_t, b3c)
    return out2d[0]
